# baseline (speedup 1.0000x reference)
import jax
import jax.numpy as jnp
from jax.experimental import pallas as pl
from jax.experimental.pallas import tpu as pltpu


def _round_up(x, m):
    return (x + m - 1) // m * m


def _head_kernel(x_ref, w_ref, b_ref, o_ref):
    h = o_ref.shape[-1]
    acc = jnp.dot(x_ref[...], w_ref[...], preferred_element_type=jnp.float32)
    o_ref[...] = (acc + b_ref[...])[:, :h].astype(o_ref.dtype)


def kernel(x, w_t, b):
    H = 336
    B, N, F, P = x.shape
    nf = F * P
    nf_w, H_pad = w_t.shape
    M = B * N
    out_dtype = x.dtype

    x_flat = x.reshape(M, nf)

    tm = 224
    if M % tm != 0:
        tm = max(8, _round_up(pl.cdiv(M, max(2, pl.cdiv(M, 256))), 8))
    grid_m = pl.cdiv(M, tm)

    x_isz = jnp.dtype(x.dtype).itemsize
    o_isz = jnp.dtype(out_dtype).itemsize
    need = (2 * tm * nf * x_isz
            + nf * H_pad * 4
            + 8 * H_pad * 4
            + 2 * tm * _round_up(H, 128) * o_isz)
    vmem_limit = int(min(need + (8 << 20), 100 << 20))

    out = pl.pallas_call(
        _head_kernel,
        out_shape=jax.ShapeDtypeStruct((M, H), out_dtype),
        grid=(grid_m,),
        in_specs=[
            pl.BlockSpec((tm, nf), lambda i: (i, 0)),
            pl.BlockSpec((nf, H_pad), lambda i: (0, 0),
                         pipeline_mode=pl.Buffered(1)),
            pl.BlockSpec((1, H_pad), lambda i: (0, 0),
                         pipeline_mode=pl.Buffered(1)),
        ],
        out_specs=pl.BlockSpec((tm, H), lambda i: (i, 0)),
        compiler_params=pltpu.CompilerParams(
            dimension_semantics=("parallel",),
            vmem_limit_bytes=vmem_limit,
        ),
    )(x_flat, w_t, b)

    return out.reshape(B, N, H)


# speedup vs baseline: 1.1449x; 1.1449x over previous
import jax
import jax.numpy as jnp
from jax.experimental import pallas as pl
from jax.experimental.pallas import tpu as pltpu


def _head_kernel(x_ref, w_ref, b_ref, o_ref):
    tb, n, f, p = x_ref.shape
    h = o_ref.shape[-1]
    xf = x_ref[...].reshape(tb * n, f * p)
    acc = jnp.dot(xf, w_ref[...], preferred_element_type=jnp.float32)
    o_ref[...] = (acc + b_ref[...])[:, :h].astype(o_ref.dtype)


def kernel(x, w_t, b):
    H = 336
    B, N, F, P = x.shape
    nf = F * P
    nf_w, H_pad = w_t.shape
    M = B * N
    out_dtype = x.dtype

    tb = 16
    while B % tb != 0:
        tb //= 2
    grid_b = B // tb

    p_pad = (P + 127) // 128 * 128
    need = (2 * tb * N * F * p_pad * 4
            + nf * H_pad * 4
            + 8 * H_pad * 4
            + 2 * tb * N * ((H + 127) // 128 * 128) * 4)
    vmem_limit = int(min(need + (8 << 20), 100 << 20))

    out = pl.pallas_call(
        _head_kernel,
        out_shape=jax.ShapeDtypeStruct((M, H), out_dtype),
        grid=(grid_b,),
        in_specs=[
            pl.BlockSpec((tb, N, F, P), lambda i: (i, 0, 0, 0)),
            pl.BlockSpec((nf, H_pad), lambda i: (0, 0),
                         pipeline_mode=pl.Buffered(1)),
            pl.BlockSpec((1, H_pad), lambda i: (0, 0),
                         pipeline_mode=pl.Buffered(1)),
        ],
        out_specs=pl.BlockSpec((tb * N, H), lambda i: (i, 0)),
        compiler_params=pltpu.CompilerParams(
            dimension_semantics=("parallel",),
            vmem_limit_bytes=vmem_limit,
        ),
    )(x, w_t, b)

    return out.reshape(B, N, H)


# speedup vs baseline: 2.6721x; 2.3340x over previous
import jax
import jax.numpy as jnp
from jax.experimental import pallas as pl
from jax.experimental.pallas import tpu as pltpu


def _head_kernel(x_ref, w_ref, b_ref, o_ref):
    tb, n, p, f = x_ref.shape
    h = o_ref.shape[-1]
    xf = x_ref[...].reshape(tb * n, p * f)
    acc = jnp.dot(xf, w_ref[...], preferred_element_type=jnp.float32)
    o_ref[...] = (acc + b_ref[...])[:, :h].astype(o_ref.dtype)


def kernel(x, w_t, b):
    H = 336
    B, N, F, P = x.shape
    nf = F * P
    nf_w, H_pad = w_t.shape
    M = B * N
    out_dtype = x.dtype

    xt = jnp.transpose(x, (0, 1, 3, 2))
    wp = w_t.reshape(F, P, H_pad).transpose(1, 0, 2).reshape(nf, H_pad)

    tb = 16
    while B % tb != 0:
        tb //= 2
    grid_b = B // tb

    need = (2 * tb * N * P * F * 4
            + nf * H_pad * 4
            + 8 * H_pad * 4
            + 2 * tb * N * ((H + 127) // 128 * 128) * 4)
    vmem_limit = int(min(need + (8 << 20), 100 << 20))

    out = pl.pallas_call(
        _head_kernel,
        out_shape=jax.ShapeDtypeStruct((M, H), out_dtype),
        grid=(grid_b,),
        in_specs=[
            pl.BlockSpec((tb, N, P, F), lambda i: (i, 0, 0, 0)),
            pl.BlockSpec((nf, H_pad), lambda i: (0, 0),
                         pipeline_mode=pl.Buffered(1)),
            pl.BlockSpec((1, H_pad), lambda i: (0, 0),
                         pipeline_mode=pl.Buffered(1)),
        ],
        out_specs=pl.BlockSpec((tb * N, H), lambda i: (i, 0)),
        compiler_params=pltpu.CompilerParams(
            dimension_semantics=("parallel",),
            vmem_limit_bytes=vmem_limit,
        ),
    )(xt, wp, b)

    return out.reshape(B, N, H)


# speedup vs baseline: 4.9230x; 1.8424x over previous
import jax
import jax.numpy as jnp
from jax.experimental import pallas as pl
from jax.experimental.pallas import tpu as pltpu


def _head_kernel(x_ref, w_ref, b_ref, o_ref, wp_ref):
    tb, n, p, f = x_ref.shape
    h = o_ref.shape[-1]
    nf = p * f
    h_pad = w_ref.shape[-1]

    @pl.when(pl.program_id(1) == 0)
    def _():
        w3 = w_ref[...].reshape(f, p, h_pad)
        wp_ref[...] = (
            w3.transpose(1, 0, 2).reshape(nf, h_pad).astype(jnp.bfloat16)
        )

    xf = x_ref[...].reshape(tb * n, nf).astype(jnp.bfloat16)
    acc = jnp.dot(xf, wp_ref[...], preferred_element_type=jnp.float32)
    res = acc + b_ref[...]
    o_ref[...] = res[:, :h].reshape(tb, n, h).astype(o_ref.dtype)


def kernel(x, w_t, b):
    H = 336
    B, N, F, P = x.shape
    nf = F * P
    nf_w, H_pad = w_t.shape
    out_dtype = x.dtype

    xt = jnp.transpose(x, (0, 1, 3, 2))

    tb = 16
    while B % (2 * tb) != 0:
        tb //= 2
    grid_j = B // (2 * tb)

    need = (2 * tb * N * P * F * 4
            + nf * H_pad * 4
            + nf * H_pad * 2
            + 8 * H_pad * 4
            + 2 * tb * 8 * ((H + 127) // 128 * 128) * 4)
    vmem_limit = int(min(need + (8 << 20), 100 << 20))

    return pl.pallas_call(
        _head_kernel,
        out_shape=jax.ShapeDtypeStruct((B, N, H), out_dtype),
        grid=(2, grid_j),
        in_specs=[
            pl.BlockSpec((tb, N, P, F),
                         lambda c, j: (c * grid_j + j, 0, 0, 0)),
            pl.BlockSpec((nf, H_pad), lambda c, j: (0, 0),
                         pipeline_mode=pl.Buffered(1)),
            pl.BlockSpec((1, H_pad), lambda c, j: (0, 0),
                         pipeline_mode=pl.Buffered(1)),
        ],
        out_specs=pl.BlockSpec((tb, N, H), lambda c, j: (c * grid_j + j, 0, 0)),
        scratch_shapes=[pltpu.VMEM((nf, H_pad), jnp.bfloat16)],
        compiler_params=pltpu.CompilerParams(
            dimension_semantics=("parallel", "arbitrary"),
            vmem_limit_bytes=vmem_limit,
        ),
    )(xt, w_t, b)


# speedup vs baseline: 5.5480x; 1.1269x over previous
import jax
import jax.numpy as jnp
from jax.experimental import pallas as pl
from jax.experimental.pallas import tpu as pltpu


def _head_kernel(x_ref, w_ref, b_ref, o_ref, wp_ref):
    tb, n, p, f = x_ref.shape
    h = o_ref.shape[-1]
    nf = p * f
    h_pad = w_ref.shape[-1]

    @pl.when(pl.program_id(1) == 0)
    def _():
        w3 = w_ref[...].reshape(f, p, h_pad)
        wp_ref[...] = (
            w3.transpose(1, 0, 2).reshape(nf, h_pad).astype(jnp.bfloat16)
        )

    xf = x_ref[...].reshape(tb * n, nf).astype(jnp.bfloat16)
    acc = jnp.dot(xf, wp_ref[...], preferred_element_type=jnp.float32)
    res = acc + b_ref[...]
    o_ref[...] = res[:, :h].reshape(tb, n, h).astype(o_ref.dtype)


def kernel(x, w_t, b):
    H = 336
    B, N, F, P = x.shape
    nf = F * P
    nf_w, H_pad = w_t.shape
    out_dtype = x.dtype

    xt = jnp.transpose(x, (0, 1, 3, 2))

    tb = 32
    while B % (2 * tb) != 0:
        tb //= 2
    grid_j = B // (2 * tb)

    need = (2 * tb * N * P * F * 4
            + nf * H_pad * 4
            + nf * H_pad * 2
            + 8 * H_pad * 4
            + 2 * tb * 8 * ((H + 127) // 128 * 128) * 4)
    vmem_limit = int(min(need + (8 << 20), 100 << 20))

    return pl.pallas_call(
        _head_kernel,
        out_shape=jax.ShapeDtypeStruct((B, N, H), out_dtype),
        grid=(2, grid_j),
        in_specs=[
            pl.BlockSpec((tb, N, P, F),
                         lambda c, j: (c * grid_j + j, 0, 0, 0)),
            pl.BlockSpec((nf, H_pad), lambda c, j: (0, 0),
                         pipeline_mode=pl.Buffered(1)),
            pl.BlockSpec((1, H_pad), lambda c, j: (0, 0),
                         pipeline_mode=pl.Buffered(1)),
        ],
        out_specs=pl.BlockSpec((tb, N, H), lambda c, j: (c * grid_j + j, 0, 0)),
        scratch_shapes=[pltpu.VMEM((nf, H_pad), jnp.bfloat16)],
        compiler_params=pltpu.CompilerParams(
            dimension_semantics=("parallel", "arbitrary"),
            vmem_limit_bytes=vmem_limit,
        ),
    )(xt, w_t, b)


# speedup vs baseline: 5.6749x; 1.0229x over previous
import jax
import jax.numpy as jnp
from jax.experimental import pallas as pl
from jax.experimental.pallas import tpu as pltpu


def _head_kernel(x_ref, w_ref, b_ref, o_ref, wp_ref):
    tb, n, p, f = x_ref.shape
    h = o_ref.shape[-1]
    nf = p * f
    h_pad = w_ref.shape[-1]

    @pl.when(pl.program_id(1) == 0)
    def _():
        w3 = w_ref[...].reshape(f, p, h_pad)
        wp_ref[...] = (
            w3.transpose(1, 0, 2).reshape(nf, h_pad).astype(jnp.bfloat16)
        )

    xf = x_ref[...].reshape(tb * n, nf).astype(jnp.bfloat16)
    acc = jnp.dot(xf, wp_ref[...], preferred_element_type=jnp.float32)
    res = acc + b_ref[...]
    o_ref[...] = res[:, :h].reshape(tb, n, h).astype(o_ref.dtype)


def kernel(x, w_t, b):
    H = 336
    B, N, F, P = x.shape
    nf = F * P
    nf_w, H_pad = w_t.shape
    out_dtype = x.dtype

    xt = jnp.transpose(x, (0, 1, 3, 2))

    tb = 64
    while B % (2 * tb) != 0:
        tb //= 2
    grid_j = B // (2 * tb)

    need = (2 * tb * N * P * F * 4
            + nf * H_pad * 4
            + nf * H_pad * 2
            + 8 * H_pad * 4
            + 2 * tb * 8 * ((H + 127) // 128 * 128) * 4)
    vmem_limit = int(min(need + (8 << 20), 100 << 20))

    return pl.pallas_call(
        _head_kernel,
        out_shape=jax.ShapeDtypeStruct((B, N, H), out_dtype),
        grid=(2, grid_j),
        in_specs=[
            pl.BlockSpec((tb, N, P, F),
                         lambda c, j: (c * grid_j + j, 0, 0, 0)),
            pl.BlockSpec((nf, H_pad), lambda c, j: (0, 0),
                         pipeline_mode=pl.Buffered(1)),
            pl.BlockSpec((1, H_pad), lambda c, j: (0, 0),
                         pipeline_mode=pl.Buffered(1)),
        ],
        out_specs=pl.BlockSpec((tb, N, H), lambda c, j: (c * grid_j + j, 0, 0)),
        scratch_shapes=[pltpu.VMEM((nf, H_pad), jnp.bfloat16)],
        compiler_params=pltpu.CompilerParams(
            dimension_semantics=("parallel", "arbitrary"),
            vmem_limit_bytes=vmem_limit,
        ),
    )(xt, w_t, b)


# speedup vs baseline: 5.6874x; 1.0022x over previous
import jax
import jax.numpy as jnp
from jax.experimental import pallas as pl
from jax.experimental.pallas import tpu as pltpu


def _head_kernel(x_ref, w_ref, b_ref, o_ref, wp_ref):
    tb, n, p, f = x_ref.shape
    h = o_ref.shape[-1]
    nf = p * f
    h_pad = w_ref.shape[-1]

    @pl.when(pl.program_id(1) == 0)
    def _():
        w3 = w_ref[...].reshape(f, p, h_pad)
        wp_ref[...] = (
            w3.transpose(1, 0, 2).reshape(nf, h_pad).astype(jnp.bfloat16)
        )

    xf = x_ref[...].reshape(tb * n, nf).astype(jnp.bfloat16)
    acc = jnp.dot(xf, wp_ref[...], preferred_element_type=jnp.float32)
    res = acc + b_ref[...]
    o_ref[...] = res[:, :h].reshape(tb, n, h).astype(o_ref.dtype)


def kernel(x, w_t, b):
    H = 336
    B, N, F, P = x.shape
    nf = F * P
    nf_w, H_pad = w_t.shape
    out_dtype = x.dtype

    xt = jnp.transpose(x, (0, 1, 3, 2))

    tb = 64
    while B % (2 * tb) != 0:
        tb //= 2
    grid_j = B // (2 * tb)

    need = (2 * tb * N * P * F * 4
            + nf * H_pad * 4
            + nf * H_pad * 2
            + 8 * H_pad * 4
            + 2 * tb * 8 * ((H + 127) // 128 * 128) * 4)
    vmem_limit = int(min(need + (8 << 20), 100 << 20))

    return pl.pallas_call(
        _head_kernel,
        out_shape=jax.ShapeDtypeStruct((B, N, H), out_dtype),
        grid=(2, grid_j),
        in_specs=[
            pl.BlockSpec((tb, N, P, F),
                         lambda c, j: (c * grid_j + j, 0, 0, 0)),
            pl.BlockSpec((nf, H_pad), lambda c, j: (0, 0)),
            pl.BlockSpec((1, H_pad), lambda c, j: (0, 0)),
        ],
        out_specs=pl.BlockSpec((tb, N, H), lambda c, j: (c * grid_j + j, 0, 0)),
        scratch_shapes=[pltpu.VMEM((nf, H_pad), jnp.bfloat16)],
        compiler_params=pltpu.CompilerParams(
            dimension_semantics=("parallel", "arbitrary"),
            vmem_limit_bytes=vmem_limit,
        ),
    )(xt, w_t, b)
